# baseline (speedup 1.0000x reference)
import functools

import jax
import jax.numpy as jnp
from jax import lax
from jax.experimental import pallas as pl
from jax.experimental.pallas import tpu as pltpu
from jax.experimental.pallas import tpu_sc as plsc

N = 10000
E = 320000
F_IN = 128
H = 64
C = 7

NC = 2
NS = 16
NW = NC * NS
CHUNK = 128
CH = -(-E // (NW * CHUNK))
E_PAD = NW * CH * CHUNK
TOT = E_PAD // CHUNK
N_PAD = ((N + 1 + NS * 8 - 1) // (NS * 8)) * (NS * 8)
RPT = N_PAD // NS
DCOL = 16
NBUF = 6
LOOKAHEAD = NBUF - 2
CH0_L1 = 54
CH0_L2 = 68

def _build_agg(mesh, D, ch0, with_deg):
    ch1 = 2 * CH - ch0
    chm = max(ch0, ch1)
    base1 = NS * ch0

    out_type = [jax.ShapeDtypeStruct((NC, N_PAD, D), jnp.float32)]
    scratch = [
        pltpu.VMEM((chm, CHUNK), jnp.int32),
        pltpu.VMEM((chm, CHUNK), jnp.int32),
        pltpu.VMEM((NBUF, CHUNK, D), jnp.float32),
        pltpu.VMEM_SHARED((N_PAD, D), jnp.float32),
        pltpu.SemaphoreType.DMA((NBUF,)),
        pltpu.SemaphoreType.DMA((NBUF,)),
    ]
    if with_deg:
        out_type.append(jax.ShapeDtypeStruct((NC, N_PAD, DCOL), jnp.float32))
        scratch += [
            pltpu.VMEM((CHUNK, DCOL), jnp.float32),
            pltpu.VMEM_SHARED((N_PAD, DCOL), jnp.float32),
            pltpu.SemaphoreType.DMA((2,)),
        ]

    @functools.partial(
        pl.kernel,
        mesh=mesh,
        compiler_params=pltpu.CompilerParams(use_tc_tiling_on_sc=False),
        out_type=tuple(out_type) if with_deg else out_type[0],
        scratch_types=tuple(scratch),
    )
    def agg(*refs):
        if with_deg:
            (z_hbm, src_hbm, dst_hbm, zeros_hbm, zeros_d_hbm, ones_hbm,
             out_hbm, deg_hbm,
             src_v, dst_v, rows_v, acc_sh, gsem, ssem,
             ones_v, deg_sh, osem) = refs
        else:
            (z_hbm, src_hbm, dst_hbm, zeros_hbm,
             out_hbm,
             src_v, dst_v, rows_v, acc_sh, gsem, ssem) = refs
        cid = lax.axis_index("c")
        sid = lax.axis_index("s")
        r0 = sid * RPT
        pltpu.sync_copy(zeros_hbm.at[pl.ds(r0, RPT)],
                        acc_sh.at[pl.ds(r0, RPT)])
        if with_deg:
            pltpu.sync_copy(zeros_d_hbm.at[pl.ds(r0, RPT)],
                            deg_sh.at[pl.ds(r0, RPT)])
            pltpu.sync_copy(ones_hbm, ones_v)

        @pl.when(cid == 0)
        def _():
            pltpu.sync_copy(src_hbm.at[pl.ds(sid * ch0, ch0)],
                            src_v.at[pl.ds(0, ch0)])
            pltpu.sync_copy(dst_hbm.at[pl.ds(sid * ch0, ch0)],
                            dst_v.at[pl.ds(0, ch0)])

        @pl.when(cid == 1)
        def _():
            pltpu.sync_copy(src_hbm.at[pl.ds(base1 + sid * ch1, ch1)],
                            src_v.at[pl.ds(0, ch1)])
            pltpu.sync_copy(dst_hbm.at[pl.ds(base1 + sid * ch1, ch1)],
                            dst_v.at[pl.ds(0, ch1)])

        nch = jnp.where(cid == 0, ch0, ch1)
        plsc.subcore_barrier()

        def gather(j):
            return pltpu.make_async_copy(
                z_hbm.at[src_v.at[j]], rows_v.at[lax.rem(j, NBUF)],
                gsem.at[lax.rem(j, NBUF)])

        def scat(j):
            return pltpu.make_async_copy(
                rows_v.at[lax.rem(j, NBUF)], acc_sh.at[dst_v.at[j]],
                ssem.at[lax.rem(j, NBUF)])

        def deg_scat(j):
            return pltpu.make_async_copy(
                ones_v, deg_sh.at[dst_v.at[j]], osem.at[lax.rem(j, 2)])

        for b in range(LOOKAHEAD):
            gather(b).start()

        def body(j, carry):
            gather(j).wait()
            scat(j).start(add=True)

            @pl.when(j >= 2)
            def _():
                scat(j - 2).wait()

            @pl.when(j + LOOKAHEAD < nch)
            def _():
                gather(j + LOOKAHEAD).start()

            if with_deg:
                @pl.when(j >= 2)
                def _():
                    deg_scat(j - 2).wait()

                deg_scat(j).start(add=True)
            return carry

        lax.fori_loop(0, nch, body, 0)
        for t in (2, 1):
            scat(nch - t).wait()
            if with_deg:
                deg_scat(nch - t).wait()
        plsc.subcore_barrier()
        pltpu.sync_copy(acc_sh.at[pl.ds(r0, RPT)],
                        out_hbm.at[cid, pl.ds(r0, RPT)])
        if with_deg:
            pltpu.sync_copy(deg_sh.at[pl.ds(r0, RPT)],
                            deg_hbm.at[cid, pl.ds(r0, RPT)])

    return agg


@functools.lru_cache(maxsize=None)
def _sc_kernels():
    mesh = plsc.VectorSubcoreMesh(core_axis_name="c", subcore_axis_name="s")
    sc_agg1 = _build_agg(mesh, H, CH0_L1, with_deg=True)
    sc_agg2 = _build_agg(mesh, DCOL, CH0_L2, with_deg=False)
    return sc_agg1, sc_agg2


_RB = 2000


def _tc_pre_body(x_ref, wl_ref, wr_ref, z_ref, y_ref):
    x = x_ref[...]
    dn = (((1,), (1,)), ((), ()))
    z_ref[...] = lax.dot_general(x, wl_ref[...], dn,
                                 preferred_element_type=jnp.float32)
    y_ref[...] = lax.dot_general(x, wr_ref[...], dn,
                                 preferred_element_type=jnp.float32)


def _tc_pre(x, w1l, w1r):
    return pl.pallas_call(
        _tc_pre_body,
        grid=(N // _RB,),
        in_specs=[
            pl.BlockSpec((_RB, F_IN), lambda i: (i, 0)),
            pl.BlockSpec((H, F_IN), lambda i: (0, 0)),
            pl.BlockSpec((H, F_IN), lambda i: (0, 0)),
        ],
        out_specs=[
            pl.BlockSpec((_RB, H), lambda i: (i, 0)),
            pl.BlockSpec((_RB, H), lambda i: (i, 0)),
        ],
        out_shape=[
            jax.ShapeDtypeStruct((N, H), jnp.float32),
            jax.ShapeDtypeStruct((N, H), jnp.float32),
        ],
    )(x, w1l, w1r)


def _tc_mid_body(p_ref, d_ref, y1_ref, b1_ref, w2l_ref, w2r_ref,
                 z2_ref, y2_ref):
    agg = p_ref[0] + p_ref[1]
    deg = d_ref[0][:, 0:1] + d_ref[1][:, 0:1]
    dinv = 1.0 / jnp.maximum(deg, 1.0)
    h = jnp.maximum(agg * dinv + b1_ref[...] + y1_ref[...], 0.0)
    dn = (((1,), (1,)), ((), ()))
    z2_ref[...] = lax.dot_general(h, w2l_ref[...], dn,
                                  preferred_element_type=jnp.float32)
    y2_ref[...] = lax.dot_general(h, w2r_ref[...], dn,
                                  preferred_element_type=jnp.float32)


def _tc_mid(p, d, y1, b1, w2l_p, w2r_p):
    return pl.pallas_call(
        _tc_mid_body,
        grid=(N // _RB,),
        in_specs=[
            pl.BlockSpec((NC, _RB, H), lambda i: (0, i, 0)),
            pl.BlockSpec((NC, _RB, DCOL), lambda i: (0, i, 0)),
            pl.BlockSpec((_RB, H), lambda i: (i, 0)),
            pl.BlockSpec((1, H), lambda i: (0, 0)),
            pl.BlockSpec((DCOL, H), lambda i: (0, 0)),
            pl.BlockSpec((DCOL, H), lambda i: (0, 0)),
        ],
        out_specs=[
            pl.BlockSpec((_RB, DCOL), lambda i: (i, 0)),
            pl.BlockSpec((_RB, DCOL), lambda i: (i, 0)),
        ],
        out_shape=[
            jax.ShapeDtypeStruct((N, DCOL), jnp.float32),
            jax.ShapeDtypeStruct((N, DCOL), jnp.float32),
        ],
    )(p, d, y1, b1, w2l_p, w2r_p)


def _tc_post_body(q_ref, d_ref, y2_ref, b2_ref, out_ref):
    s = q_ref[0] + q_ref[1]
    deg = d_ref[0][:, 0:1] + d_ref[1][:, 0:1]
    dinv = 1.0 / jnp.maximum(deg, 1.0)
    o = (s * dinv + b2_ref[...] + y2_ref[...])[:, :C]
    m = jnp.max(o, axis=1, keepdims=True)
    ex = jnp.exp(o - m)
    lse = jnp.log(jnp.sum(ex, axis=1, keepdims=True))
    out_ref[...] = o - m - lse


def _tc_post(q, d, y2, b2_p):
    return pl.pallas_call(
        _tc_post_body,
        grid=(N // _RB,),
        in_specs=[
            pl.BlockSpec((NC, _RB, DCOL), lambda i: (0, i, 0)),
            pl.BlockSpec((NC, _RB, DCOL), lambda i: (0, i, 0)),
            pl.BlockSpec((_RB, DCOL), lambda i: (i, 0)),
            pl.BlockSpec((1, DCOL), lambda i: (0, 0)),
        ],
        out_specs=pl.BlockSpec((_RB, C), lambda i: (i, 0)),
        out_shape=jax.ShapeDtypeStruct((N, C), jnp.float32),
    )(q, d, y2, b2_p)


def kernel(x, edge_index, W1_l, W1_r, b1, W2_l, W2_r, b2):
    src = edge_index[0].astype(jnp.int32)
    dst = edge_index[1].astype(jnp.int32)
    pad = E_PAD - E
    src2 = jnp.concatenate(
        [src, jnp.zeros((pad,), jnp.int32)]).reshape(TOT, CHUNK)
    dst2 = jnp.concatenate(
        [dst, jnp.full((pad,), N, jnp.int32)]).reshape(TOT, CHUNK)
    zeros_h = jnp.zeros((N_PAD, H), jnp.float32)
    zeros_d = jnp.zeros((N_PAD, DCOL), jnp.float32)
    ones = jnp.ones((CHUNK, DCOL), jnp.float32)
    w2l_p = jnp.pad(W2_l, ((0, DCOL - C), (0, 0)))
    w2r_p = jnp.pad(W2_r, ((0, DCOL - C), (0, 0)))
    b2_p = jnp.pad(b2, (0, DCOL - C)).reshape(1, DCOL)

    sc_agg1, sc_agg2 = _sc_kernels()
    z1, y1 = _tc_pre(x, W1_l, W1_r)
    p, d = sc_agg1(z1, src2, dst2, zeros_h, zeros_d, ones)
    z2, y2 = _tc_mid(p, d, y1, b1.reshape(1, H), w2l_p, w2r_p)
    q = sc_agg2(z2, src2, dst2, zeros_d)
    return _tc_post(q, d, y2, b2_p)


# speedup vs baseline: 1.1087x; 1.1087x over previous
import functools

import jax
import jax.numpy as jnp
from jax import lax
from jax.experimental import pallas as pl
from jax.experimental.pallas import tpu as pltpu
from jax.experimental.pallas import tpu_sc as plsc

N = 10000
E = 320000
F_IN = 128
H = 64
C = 7

NC = 2
NS = 16
NW = NC * NS
CHUNK = 128
CH = -(-E // (NW * CHUNK))
E_PAD = NW * CH * CHUNK
TOT = E_PAD // CHUNK
N_PAD = ((N + 1 + NS * 8 - 1) // (NS * 8)) * (NS * 8)
RPT = N_PAD // NS
DCOL = 16
NBUF = 6
LOOKAHEAD = NBUF - 2
CH0_L1 = 104
CH0_L2 = 96

def _build_agg(mesh, D, ch0, with_deg):
    ch1 = 2 * CH - ch0
    chm = max(ch0, ch1)
    base1 = NS * ch0

    out_type = [jax.ShapeDtypeStruct((NC, N_PAD, D), jnp.float32)]
    scratch = [
        pltpu.VMEM((chm, CHUNK), jnp.int32),
        pltpu.VMEM((chm, CHUNK), jnp.int32),
        pltpu.VMEM((NBUF, CHUNK, D), jnp.float32),
        pltpu.VMEM_SHARED((N_PAD, D), jnp.float32),
        pltpu.SemaphoreType.DMA((NBUF,)),
        pltpu.SemaphoreType.DMA((NBUF,)),
    ]
    if with_deg:
        out_type.append(jax.ShapeDtypeStruct((NC, N_PAD, DCOL), jnp.float32))
        scratch += [
            pltpu.VMEM((CHUNK, DCOL), jnp.float32),
            pltpu.VMEM_SHARED((N_PAD, DCOL), jnp.float32),
            pltpu.SemaphoreType.DMA((2,)),
        ]

    @functools.partial(
        pl.kernel,
        mesh=mesh,
        compiler_params=pltpu.CompilerParams(use_tc_tiling_on_sc=False),
        out_type=tuple(out_type) if with_deg else out_type[0],
        scratch_types=tuple(scratch),
    )
    def agg(*refs):
        if with_deg:
            (z_hbm, src_hbm, dst_hbm, zeros_hbm, zeros_d_hbm, ones_hbm,
             out_hbm, deg_hbm,
             src_v, dst_v, rows_v, acc_sh, gsem, ssem,
             ones_v, deg_sh, osem) = refs
        else:
            (z_hbm, src_hbm, dst_hbm, zeros_hbm,
             out_hbm,
             src_v, dst_v, rows_v, acc_sh, gsem, ssem) = refs
        cid = lax.axis_index("c")
        sid = lax.axis_index("s")
        r0 = sid * RPT
        pltpu.sync_copy(zeros_hbm.at[pl.ds(r0, RPT)],
                        acc_sh.at[pl.ds(r0, RPT)])
        if with_deg:
            pltpu.sync_copy(zeros_d_hbm.at[pl.ds(r0, RPT)],
                            deg_sh.at[pl.ds(r0, RPT)])
            pltpu.sync_copy(ones_hbm, ones_v)

        @pl.when(cid == 0)
        def _():
            pltpu.sync_copy(src_hbm.at[pl.ds(sid * ch0, ch0)],
                            src_v.at[pl.ds(0, ch0)])
            pltpu.sync_copy(dst_hbm.at[pl.ds(sid * ch0, ch0)],
                            dst_v.at[pl.ds(0, ch0)])

        @pl.when(cid == 1)
        def _():
            pltpu.sync_copy(src_hbm.at[pl.ds(base1 + sid * ch1, ch1)],
                            src_v.at[pl.ds(0, ch1)])
            pltpu.sync_copy(dst_hbm.at[pl.ds(base1 + sid * ch1, ch1)],
                            dst_v.at[pl.ds(0, ch1)])

        nch = jnp.where(cid == 0, ch0, ch1)
        plsc.subcore_barrier()

        def gather(j):
            return pltpu.make_async_copy(
                z_hbm.at[src_v.at[j]], rows_v.at[lax.rem(j, NBUF)],
                gsem.at[lax.rem(j, NBUF)])

        def scat(j):
            return pltpu.make_async_copy(
                rows_v.at[lax.rem(j, NBUF)], acc_sh.at[dst_v.at[j]],
                ssem.at[lax.rem(j, NBUF)])

        def deg_scat(j):
            return pltpu.make_async_copy(
                ones_v, deg_sh.at[dst_v.at[j]], osem.at[lax.rem(j, 2)])

        for b in range(LOOKAHEAD):
            gather(b).start()

        def body(j, carry):
            gather(j).wait()
            scat(j).start(add=True)

            @pl.when(j >= 2)
            def _():
                scat(j - 2).wait()

            @pl.when(j + LOOKAHEAD < nch)
            def _():
                gather(j + LOOKAHEAD).start()

            if with_deg:
                @pl.when(j >= 2)
                def _():
                    deg_scat(j - 2).wait()

                deg_scat(j).start(add=True)
            return carry

        lax.fori_loop(0, nch, body, 0)
        for t in (2, 1):
            scat(nch - t).wait()
            if with_deg:
                deg_scat(nch - t).wait()
        plsc.subcore_barrier()
        pltpu.sync_copy(acc_sh.at[pl.ds(r0, RPT)],
                        out_hbm.at[cid, pl.ds(r0, RPT)])
        if with_deg:
            pltpu.sync_copy(deg_sh.at[pl.ds(r0, RPT)],
                            deg_hbm.at[cid, pl.ds(r0, RPT)])

    return agg


@functools.lru_cache(maxsize=None)
def _sc_kernels():
    mesh = plsc.VectorSubcoreMesh(core_axis_name="c", subcore_axis_name="s")
    sc_agg1 = _build_agg(mesh, H, CH0_L1, with_deg=True)
    sc_agg2 = _build_agg(mesh, DCOL, CH0_L2, with_deg=False)
    return sc_agg1, sc_agg2


_RB = 2000


def _tc_pre_body(x_ref, wl_ref, wr_ref, z_ref, y_ref):
    x = x_ref[...]
    dn = (((1,), (1,)), ((), ()))
    z_ref[...] = lax.dot_general(x, wl_ref[...], dn,
                                 preferred_element_type=jnp.float32)
    y_ref[...] = lax.dot_general(x, wr_ref[...], dn,
                                 preferred_element_type=jnp.float32)


def _tc_pre(x, w1l, w1r):
    return pl.pallas_call(
        _tc_pre_body,
        grid=(N // _RB,),
        in_specs=[
            pl.BlockSpec((_RB, F_IN), lambda i: (i, 0)),
            pl.BlockSpec((H, F_IN), lambda i: (0, 0)),
            pl.BlockSpec((H, F_IN), lambda i: (0, 0)),
        ],
        out_specs=[
            pl.BlockSpec((_RB, H), lambda i: (i, 0)),
            pl.BlockSpec((_RB, H), lambda i: (i, 0)),
        ],
        out_shape=[
            jax.ShapeDtypeStruct((N, H), jnp.float32),
            jax.ShapeDtypeStruct((N, H), jnp.float32),
        ],
    )(x, w1l, w1r)


def _tc_mid_body(p_ref, d_ref, y1_ref, b1_ref, w2l_ref, w2r_ref,
                 z2_ref, y2_ref):
    agg = p_ref[0] + p_ref[1]
    deg = d_ref[0][:, 0:1] + d_ref[1][:, 0:1]
    dinv = 1.0 / jnp.maximum(deg, 1.0)
    h = jnp.maximum(agg * dinv + b1_ref[...] + y1_ref[...], 0.0)
    dn = (((1,), (1,)), ((), ()))
    z2_ref[...] = lax.dot_general(h, w2l_ref[...], dn,
                                  preferred_element_type=jnp.float32)
    y2_ref[...] = lax.dot_general(h, w2r_ref[...], dn,
                                  preferred_element_type=jnp.float32)


def _tc_mid(p, d, y1, b1, w2l_p, w2r_p):
    return pl.pallas_call(
        _tc_mid_body,
        grid=(N // _RB,),
        in_specs=[
            pl.BlockSpec((NC, _RB, H), lambda i: (0, i, 0)),
            pl.BlockSpec((NC, _RB, DCOL), lambda i: (0, i, 0)),
            pl.BlockSpec((_RB, H), lambda i: (i, 0)),
            pl.BlockSpec((1, H), lambda i: (0, 0)),
            pl.BlockSpec((DCOL, H), lambda i: (0, 0)),
            pl.BlockSpec((DCOL, H), lambda i: (0, 0)),
        ],
        out_specs=[
            pl.BlockSpec((_RB, DCOL), lambda i: (i, 0)),
            pl.BlockSpec((_RB, DCOL), lambda i: (i, 0)),
        ],
        out_shape=[
            jax.ShapeDtypeStruct((N, DCOL), jnp.float32),
            jax.ShapeDtypeStruct((N, DCOL), jnp.float32),
        ],
    )(p, d, y1, b1, w2l_p, w2r_p)


def _tc_post_body(q_ref, d_ref, y2_ref, b2_ref, out_ref):
    s = q_ref[0] + q_ref[1]
    deg = d_ref[0][:, 0:1] + d_ref[1][:, 0:1]
    dinv = 1.0 / jnp.maximum(deg, 1.0)
    o = (s * dinv + b2_ref[...] + y2_ref[...])[:, :C]
    m = jnp.max(o, axis=1, keepdims=True)
    ex = jnp.exp(o - m)
    lse = jnp.log(jnp.sum(ex, axis=1, keepdims=True))
    out_ref[...] = o - m - lse


def _tc_post(q, d, y2, b2_p):
    return pl.pallas_call(
        _tc_post_body,
        grid=(N // _RB,),
        in_specs=[
            pl.BlockSpec((NC, _RB, DCOL), lambda i: (0, i, 0)),
            pl.BlockSpec((NC, _RB, DCOL), lambda i: (0, i, 0)),
            pl.BlockSpec((_RB, DCOL), lambda i: (i, 0)),
            pl.BlockSpec((1, DCOL), lambda i: (0, 0)),
        ],
        out_specs=pl.BlockSpec((_RB, C), lambda i: (i, 0)),
        out_shape=jax.ShapeDtypeStruct((N, C), jnp.float32),
    )(q, d, y2, b2_p)


def kernel(x, edge_index, W1_l, W1_r, b1, W2_l, W2_r, b2):
    src = edge_index[0].astype(jnp.int32)
    dst = edge_index[1].astype(jnp.int32)
    pad = E_PAD - E
    src2 = jnp.concatenate(
        [src, jnp.zeros((pad,), jnp.int32)]).reshape(TOT, CHUNK)
    dst2 = jnp.concatenate(
        [dst, jnp.full((pad,), N, jnp.int32)]).reshape(TOT, CHUNK)
    zeros_h = jnp.zeros((N_PAD, H), jnp.float32)
    zeros_d = jnp.zeros((N_PAD, DCOL), jnp.float32)
    ones = jnp.ones((CHUNK, DCOL), jnp.float32)
    w2l_p = jnp.pad(W2_l, ((0, DCOL - C), (0, 0)))
    w2r_p = jnp.pad(W2_r, ((0, DCOL - C), (0, 0)))
    b2_p = jnp.pad(b2, (0, DCOL - C)).reshape(1, DCOL)

    sc_agg1, sc_agg2 = _sc_kernels()
    z1, y1 = _tc_pre(x, W1_l, W1_r)
    p, d = sc_agg1(z1, src2, dst2, zeros_h, zeros_d, ones)
    z2, y2 = _tc_mid(p, d, y1, b1.reshape(1, H), w2l_p, w2r_p)
    q = sc_agg2(z2, src2, dst2, zeros_d)
    return _tc_post(q, d, y2, b2_p)
